# baseline (speedup 1.0000x reference)
import jax
import jax.numpy as jnp
from jax.experimental import pallas as pl
from jax.experimental.pallas import tpu as pltpu

BM = 200
NBUF = 4
_BN_RSQRT = (1.0 + 1e-5) ** -0.5


def _mlp(pooled, w1_ref, b1_ref, g1_ref, bt1_ref, w2_ref, b2_ref, g2_ref,
         bt2_ref):
    s1 = g1_ref[...] * _BN_RSQRT
    t = jax.lax.dot_general(
        pooled, w1_ref[...], (((1,), (0,)), ((), ())),
        preferred_element_type=jnp.float32)
    t = jnp.maximum(t * s1 + (b1_ref[...] * s1 + bt1_ref[...]), 0.0)
    s2 = g2_ref[...] * _BN_RSQRT
    t = jax.lax.dot_general(
        t, w2_ref[...], (((1,), (0,)), ((), ())),
        preferred_element_type=jnp.float32)
    return jnp.maximum(t * s2 + (b2_ref[...] * s2 + bt2_ref[...]), 0.0)


def _make_kernel(n, d, nb):
    adj_spec = pl.BlockSpec(
        (BM, n), lambda i: (i, 0),
        pipeline_mode=pl.Buffered(buffer_count=NBUF, use_lookahead=True))
    abf_out_spec = pl.BlockSpec((BM, n), lambda i: (i, 0))
    abf_in_spec = pl.BlockSpec(
        (BM, n), lambda i: (i, 0),
        pipeline_mode=pl.Buffered(buffer_count=NBUF, use_lookahead=True))

    def body(adj_hbm, h0_ref, eps_ref,
             w10_ref, b10_ref, g10_ref, bt10_ref,
             w20_ref, b20_ref, g20_ref, bt20_ref,
             w11_ref, b11_ref, g11_ref, bt11_ref,
             w21_ref, b21_ref, g21_ref, bt21_ref,
             wp_ref, bp_ref, out_ref, abf_hbm, h1_ref):

        def phase0(adj_blk, abf_blk):
            i = pl.program_id(0)
            a = adj_blk[...]
            abf_blk[...] = a.astype(jnp.bfloat16)
            pooled = jax.lax.dot_general(
                a, h0_ref[...], (((1,), (0,)), ((), ())),
                preferred_element_type=jnp.float32)
            pooled = pooled + (1.0 + eps_ref[0, 0]) * h0_ref[pl.ds(i * BM, BM), :]
            h1_ref[pl.ds(i * BM, BM), :] = _mlp(
                pooled, w10_ref, b10_ref, g10_ref, bt10_ref,
                w20_ref, b20_ref, g20_ref, bt20_ref).astype(jnp.bfloat16)

        pltpu.emit_pipeline(
            phase0, grid=(nb,), in_specs=[adj_spec],
            out_specs=[abf_out_spec])(adj_hbm, abf_hbm)

        def phase1(abf_blk):
            j = pl.program_id(0)
            pooled = jax.lax.dot_general(
                abf_blk[...], h1_ref[...], (((1,), (0,)), ((), ())),
                preferred_element_type=jnp.float32)
            pooled = pooled + (1.0 + eps_ref[0, 1]) * h1_ref[pl.ds(j * BM, BM), :].astype(jnp.float32)
            h2 = _mlp(pooled, w11_ref, b11_ref, g11_ref, bt11_ref,
                      w21_ref, b21_ref, g21_ref, bt21_ref)
            score = jax.lax.dot_general(
                h2, wp_ref[...], (((1,), (0,)), ((), ())),
                preferred_element_type=jnp.float32)
            out_ref[pl.ds(j * BM, BM), :] = score + bp_ref[0, 0]

        pltpu.emit_pipeline(
            phase1, grid=(nb,), in_specs=[abf_in_spec])(abf_hbm)

    return body


@jax.jit
def _run(seq1, adj, eps,
         l0_W1, l0_b1, l0_bn1_g, l0_bn1_b, l0_W2, l0_b2, l0_bn2_g, l0_bn2_b,
         l1_W1, l1_b1, l1_bn1_g, l1_bn1_b, l1_W2, l1_b2, l1_bn2_g, l1_bn2_b,
         Wp, bp):
    n, d = seq1.shape
    nb = n // BM
    r = lambda v: v.reshape(1, d)
    vmem = pl.BlockSpec(memory_space=pltpu.MemorySpace.VMEM)
    score, _ = pl.pallas_call(
        _make_kernel(n, d, nb),
        in_specs=[pl.BlockSpec(memory_space=pl.ANY)] + [vmem] * 20,
        out_specs=[vmem, pl.BlockSpec(memory_space=pl.ANY)],
        out_shape=[jax.ShapeDtypeStruct((n, 1), jnp.float32),
                   jax.ShapeDtypeStruct((n, n), jnp.bfloat16)],
        scratch_shapes=[pltpu.VMEM((n, d), jnp.bfloat16)],
    )(adj, seq1, eps.reshape(1, 2),
      l0_W1, r(l0_b1), r(l0_bn1_g), r(l0_bn1_b),
      l0_W2, r(l0_b2), r(l0_bn2_g), r(l0_bn2_b),
      l1_W1, r(l1_b1), r(l1_bn1_g), r(l1_bn1_b),
      l1_W2, r(l1_b2), r(l1_bn2_g), r(l1_bn2_b),
      Wp, bp.reshape(1, 1))
    return score


def kernel(seq1, adj, eps,
           l0_W1, l0_b1, l0_bn1_g, l0_bn1_b, l0_W2, l0_b2, l0_bn2_g, l0_bn2_b,
           l1_W1, l1_b1, l1_bn1_g, l1_bn1_b, l1_W2, l1_b2, l1_bn2_g, l1_bn2_b,
           Wp, bp):
    return _run(seq1, adj, eps,
                l0_W1, l0_b1, l0_bn1_g, l0_bn1_b, l0_W2, l0_b2, l0_bn2_g,
                l0_bn2_b, l1_W1, l1_b1, l1_bn1_g, l1_bn1_b, l1_W2, l1_b2,
                l1_bn2_g, l1_bn2_b, Wp, bp)


# speedup vs baseline: 1.0903x; 1.0903x over previous
import jax
import jax.numpy as jnp
from jax.experimental import pallas as pl
from jax.experimental.pallas import tpu as pltpu

BM = 200
NBUF = 5
_BN_RSQRT = (1.0 + 1e-5) ** -0.5


def _mlp(pooled, w1_ref, b1_ref, g1_ref, bt1_ref, w2_ref, b2_ref, g2_ref,
         bt2_ref):
    s1 = g1_ref[...] * _BN_RSQRT
    t = jax.lax.dot_general(
        pooled, w1_ref[...], (((1,), (0,)), ((), ())),
        preferred_element_type=jnp.float32)
    t = jnp.maximum(t * s1 + (b1_ref[...] * s1 + bt1_ref[...]), 0.0)
    s2 = g2_ref[...] * _BN_RSQRT
    t = jax.lax.dot_general(
        t, w2_ref[...], (((1,), (0,)), ((), ())),
        preferred_element_type=jnp.float32)
    return jnp.maximum(t * s2 + (b2_ref[...] * s2 + bt2_ref[...]), 0.0)


def _make_kernel(n, d, nb):
    adj_spec = pl.BlockSpec(
        (BM, n), lambda i: (i % nb, 0),
        pipeline_mode=pl.Buffered(buffer_count=NBUF, use_lookahead=True))

    def body(adj_hbm, h0_ref, eps_ref,
             w10_ref, b10_ref, g10_ref, bt10_ref,
             w20_ref, b20_ref, g20_ref, bt20_ref,
             w11_ref, b11_ref, g11_ref, bt11_ref,
             w21_ref, b21_ref, g21_ref, bt21_ref,
             wp_ref, bp_ref, out_ref, h1_ref):

        def step(adj_blk):
            i = pl.program_id(0)

            @pl.when(i < nb)
            def _layer0():
                pooled = jax.lax.dot_general(
                    adj_blk[...], h0_ref[...], (((1,), (0,)), ((), ())),
                    preferred_element_type=jnp.float32)
                pooled = pooled + (1.0 + eps_ref[0, 0]) * h0_ref[pl.ds(i * BM, BM), :]
                h1_ref[pl.ds(i * BM, BM), :] = _mlp(
                    pooled, w10_ref, b10_ref, g10_ref, bt10_ref,
                    w20_ref, b20_ref, g20_ref, bt20_ref)

            @pl.when(i >= nb)
            def _layer1_head():
                j = i - nb
                pooled = jax.lax.dot_general(
                    adj_blk[...], h1_ref[...], (((1,), (0,)), ((), ())),
                    preferred_element_type=jnp.float32)
                pooled = pooled + (1.0 + eps_ref[0, 1]) * h1_ref[pl.ds(j * BM, BM), :]
                h2 = _mlp(pooled, w11_ref, b11_ref, g11_ref, bt11_ref,
                          w21_ref, b21_ref, g21_ref, bt21_ref)
                score = jax.lax.dot_general(
                    h2, wp_ref[...], (((1,), (0,)), ((), ())),
                    preferred_element_type=jnp.float32)
                out_ref[pl.ds(j * BM, BM), :] = score + bp_ref[0, 0]

        pltpu.emit_pipeline(
            step, grid=(2 * nb,), in_specs=[adj_spec])(adj_hbm)

    return body


@jax.jit
def _run(seq1, adj, eps,
         l0_W1, l0_b1, l0_bn1_g, l0_bn1_b, l0_W2, l0_b2, l0_bn2_g, l0_bn2_b,
         l1_W1, l1_b1, l1_bn1_g, l1_bn1_b, l1_W2, l1_b2, l1_bn2_g, l1_bn2_b,
         Wp, bp):
    n, d = seq1.shape
    nb = n // BM
    r = lambda v: v.reshape(1, d)
    vmem = pl.BlockSpec(memory_space=pltpu.MemorySpace.VMEM)
    score = pl.pallas_call(
        _make_kernel(n, d, nb),
        in_specs=[pl.BlockSpec(memory_space=pl.ANY)] + [vmem] * 20,
        out_specs=vmem,
        out_shape=jax.ShapeDtypeStruct((n, 1), jnp.float32),
        scratch_shapes=[pltpu.VMEM((n, d), jnp.float32)],
    )(adj, seq1, eps.reshape(1, 2),
      l0_W1, r(l0_b1), r(l0_bn1_g), r(l0_bn1_b),
      l0_W2, r(l0_b2), r(l0_bn2_g), r(l0_bn2_b),
      l1_W1, r(l1_b1), r(l1_bn1_g), r(l1_bn1_b),
      l1_W2, r(l1_b2), r(l1_bn2_g), r(l1_bn2_b),
      Wp, bp.reshape(1, 1))
    return score


def kernel(seq1, adj, eps,
           l0_W1, l0_b1, l0_bn1_g, l0_bn1_b, l0_W2, l0_b2, l0_bn2_g, l0_bn2_b,
           l1_W1, l1_b1, l1_bn1_g, l1_bn1_b, l1_W2, l1_b2, l1_bn2_g, l1_bn2_b,
           Wp, bp):
    return _run(seq1, adj, eps,
                l0_W1, l0_b1, l0_bn1_g, l0_bn1_b, l0_W2, l0_b2, l0_bn2_g,
                l0_bn2_b, l1_W1, l1_b1, l1_bn1_g, l1_bn1_b, l1_W2, l1_b2,
                l1_bn2_g, l1_bn2_b, Wp, bp)


# speedup vs baseline: 1.1040x; 1.0125x over previous
import jax
import jax.numpy as jnp
from jax.experimental import pallas as pl
from jax.experimental.pallas import tpu as pltpu

BM = 400
NBUF = 3
_BN_RSQRT = (1.0 + 1e-5) ** -0.5


def _mlp(pooled, w1_ref, b1_ref, g1_ref, bt1_ref, w2_ref, b2_ref, g2_ref,
         bt2_ref):
    s1 = g1_ref[...] * _BN_RSQRT
    t = jax.lax.dot_general(
        pooled, w1_ref[...], (((1,), (0,)), ((), ())),
        preferred_element_type=jnp.float32)
    t = jnp.maximum(t * s1 + (b1_ref[...] * s1 + bt1_ref[...]), 0.0)
    s2 = g2_ref[...] * _BN_RSQRT
    t = jax.lax.dot_general(
        t, w2_ref[...], (((1,), (0,)), ((), ())),
        preferred_element_type=jnp.float32)
    return jnp.maximum(t * s2 + (b2_ref[...] * s2 + bt2_ref[...]), 0.0)


def _make_kernel(n, d, nb):
    adj_spec = pl.BlockSpec(
        (BM, n), lambda i: (i % nb, 0),
        pipeline_mode=pl.Buffered(buffer_count=NBUF, use_lookahead=True))

    def body(adj_hbm, h0_ref, eps_ref,
             w10_ref, b10_ref, g10_ref, bt10_ref,
             w20_ref, b20_ref, g20_ref, bt20_ref,
             w11_ref, b11_ref, g11_ref, bt11_ref,
             w21_ref, b21_ref, g21_ref, bt21_ref,
             wp_ref, bp_ref, out_ref, h1_ref):

        def step(adj_blk):
            i = pl.program_id(0)

            @pl.when(i < nb)
            def _layer0():
                pooled = jax.lax.dot_general(
                    adj_blk[...], h0_ref[...], (((1,), (0,)), ((), ())),
                    preferred_element_type=jnp.float32)
                pooled = pooled + (1.0 + eps_ref[0, 0]) * h0_ref[pl.ds(i * BM, BM), :]
                h1_ref[pl.ds(i * BM, BM), :] = _mlp(
                    pooled, w10_ref, b10_ref, g10_ref, bt10_ref,
                    w20_ref, b20_ref, g20_ref, bt20_ref)

            @pl.when(i >= nb)
            def _layer1_head():
                j = i - nb
                pooled = jax.lax.dot_general(
                    adj_blk[...], h1_ref[...], (((1,), (0,)), ((), ())),
                    preferred_element_type=jnp.float32)
                pooled = pooled + (1.0 + eps_ref[0, 1]) * h1_ref[pl.ds(j * BM, BM), :]
                h2 = _mlp(pooled, w11_ref, b11_ref, g11_ref, bt11_ref,
                          w21_ref, b21_ref, g21_ref, bt21_ref)
                score_row = jax.lax.dot_general(
                    wp_ref[...], h2, (((1,), (1,)), ((), ())),
                    preferred_element_type=jnp.float32)
                out_ref[pl.ds(j, 1), :] = score_row + bp_ref[0, 0]

        pltpu.emit_pipeline(
            step, grid=(2 * nb,), in_specs=[adj_spec])(adj_hbm)

    return body


@jax.jit
def _run(seq1, adj, eps,
         l0_W1, l0_b1, l0_bn1_g, l0_bn1_b, l0_W2, l0_b2, l0_bn2_g, l0_bn2_b,
         l1_W1, l1_b1, l1_bn1_g, l1_bn1_b, l1_W2, l1_b2, l1_bn2_g, l1_bn2_b,
         Wp, bp):
    n, d = seq1.shape
    nb = n // BM
    r = lambda v: v.reshape(1, d)
    vmem = pl.BlockSpec(memory_space=pltpu.MemorySpace.VMEM)
    score2d = pl.pallas_call(
        _make_kernel(n, d, nb),
        in_specs=[pl.BlockSpec(memory_space=pl.ANY)] + [vmem] * 20,
        out_specs=vmem,
        out_shape=jax.ShapeDtypeStruct((nb, BM), jnp.float32),
        scratch_shapes=[pltpu.VMEM((n, d), jnp.float32)],
    )(adj, seq1, eps.reshape(1, 2),
      l0_W1, r(l0_b1), r(l0_bn1_g), r(l0_bn1_b),
      l0_W2, r(l0_b2), r(l0_bn2_g), r(l0_bn2_b),
      l1_W1, r(l1_b1), r(l1_bn1_g), r(l1_bn1_b),
      l1_W2, r(l1_b2), r(l1_bn2_g), r(l1_bn2_b),
      Wp.reshape(1, d), bp.reshape(1, 1))
    return score2d.reshape(n, 1)


def kernel(seq1, adj, eps,
           l0_W1, l0_b1, l0_bn1_g, l0_bn1_b, l0_W2, l0_b2, l0_bn2_g, l0_bn2_b,
           l1_W1, l1_b1, l1_bn1_g, l1_bn1_b, l1_W2, l1_b2, l1_bn2_g, l1_bn2_b,
           Wp, bp):
    return _run(seq1, adj, eps,
                l0_W1, l0_b1, l0_bn1_g, l0_bn1_b, l0_W2, l0_b2, l0_bn2_g,
                l0_bn2_b, l1_W1, l1_b1, l1_bn1_g, l1_bn1_b, l1_W2, l1_b2,
                l1_bn2_g, l1_bn2_b, Wp, bp)
